# Initial kernel scaffold; baseline (speedup 1.0000x reference)
import jax
import jax.numpy as jnp
from jax.experimental import pallas as pl


def kernel(node_ids, rel_ids, center_mol_idx, non_molecule_node_ids, edge_index, node_emb_table, rel_emb_table, lin_W, lin_b, conv1_W, conv1_b, conv2_W, conv2_b, ec_W, ec_b, mp_W, mp_b, nc_W, nc_b, bp_W, bp_b):
    raise NotImplementedError("write your pallas kernel here")


# speedup vs baseline: 3.7655x; 3.7655x over previous
import functools

import jax
import jax.numpy as jnp
from jax import lax
from jax.experimental import pallas as pl
from jax.experimental.pallas import tpu as pltpu
from jax.experimental.pallas import tpu_sc as plsc

D = 128
N = 10000
NP = 10240
E = 320000
NC = 2
NS = 16
NW = NC * NS
EPW = E // NW
CH = 128
NFULL = EPW // CH
REM = EPW - NFULL * CH
ROWS_PER_TILE = NP // NS


def _mesh():
    return plsc.VectorSubcoreMesh(core_axis_name="c", subcore_axis_name="s")


def _wid():
    return lax.axis_index("s") * NC + lax.axis_index("c")


def _sc_gather_rows(table, idx, ch):
    (b,) = idx.shape
    _, d = table.shape
    bpw = b // NW
    nch = bpw // ch

    @functools.partial(
        pl.kernel,
        out_type=jax.ShapeDtypeStruct((b, d), jnp.float32),
        mesh=_mesh(),
        scratch_types=[
            pltpu.VMEM((ch,), jnp.int32),
            pltpu.VMEM((ch, d), jnp.float32),
            pltpu.SemaphoreType.DMA,
        ],
    )
    def k(table_h, idx_h, out_h, idx_v, rows_v, sem):
        base = _wid() * bpw

        def body(j, carry):
            off = pl.multiple_of(base + j * ch, 8)
            pltpu.sync_copy(idx_h.at[pl.ds(off, ch)], idx_v)
            pltpu.async_copy(table_h.at[idx_v], rows_v, sem).wait()
            pltpu.sync_copy(rows_v, out_h.at[pl.ds(off, ch)])
            return carry

        lax.fori_loop(0, nch, body, 0)

    return k(table, idx)


def _sc_msgpass(x, ea, src, dst, rel, zblk):
    @functools.partial(
        pl.kernel,
        out_type=jax.ShapeDtypeStruct((NC, NP, D), jnp.float32),
        mesh=_mesh(),
        scratch_types=[
            pltpu.VMEM((CH,), jnp.int32),
            pltpu.VMEM((CH,), jnp.int32),
            pltpu.VMEM((CH,), jnp.int32),
            pltpu.VMEM((CH, D), jnp.float32),
            pltpu.VMEM((CH, D), jnp.float32),
            pltpu.VMEM((REM,), jnp.int32),
            pltpu.VMEM((REM,), jnp.int32),
            pltpu.VMEM((REM,), jnp.int32),
            pltpu.VMEM((REM, D), jnp.float32),
            pltpu.VMEM((REM, D), jnp.float32),
            pltpu.VMEM_SHARED((NP, D), jnp.float32),
            pltpu.SemaphoreType.DMA,
            pltpu.SemaphoreType.DMA,
        ],
    )
    def k(x_h, ea_h, src_h, dst_h, rel_h, z_h, out_h,
          si, di, ri, xs, eav, si2, di2, ri2, xs2, eav2, agg, sem_a, sem_b):
        cid = lax.axis_index("c")
        sid = lax.axis_index("s")
        base = (sid * NC + cid) * EPW

        def zbody(t, carry):
            r0 = pl.multiple_of(sid * ROWS_PER_TILE + t * CH, 8)
            pltpu.sync_copy(z_h, agg.at[pl.ds(r0, CH)])
            return carry

        lax.fori_loop(0, ROWS_PER_TILE // CH, zbody, 0)
        plsc.subcore_barrier()

        def do_chunk(off, n, si_, di_, ri_, xs_, eav_):
            pltpu.sync_copy(src_h.at[pl.ds(off, n)], si_)
            pltpu.sync_copy(rel_h.at[pl.ds(off, n)], ri_)
            pltpu.sync_copy(dst_h.at[pl.ds(off, n)], di_)
            ca = pltpu.async_copy(x_h.at[si_], xs_, sem_a)
            cb = pltpu.async_copy(ea_h.at[ri_], eav_, sem_b)
            ca.wait()
            cb.wait()

            def cbody(i, carry):
                for jj in range(D // 16):
                    sl = pl.ds(jj * 16, 16)
                    xs_[i, sl] = jnp.maximum(xs_[i, sl] + eav_[i, sl], 0.0)
                return carry

            lax.fori_loop(0, n, cbody, 0)
            pltpu.sync_copy(xs_, agg.at[di_], add=True)

        def ebody(j, carry):
            do_chunk(pl.multiple_of(base + j * CH, 8), CH, si, di, ri, xs, eav)
            return carry

        lax.fori_loop(0, NFULL, ebody, 0)
        do_chunk(pl.multiple_of(base + NFULL * CH, 8), REM,
                 si2, di2, ri2, xs2, eav2)

        plsc.subcore_barrier()

        def obody(t, carry):
            r0 = pl.multiple_of(sid * ROWS_PER_TILE + t * CH, 8)
            pltpu.sync_copy(agg.at[pl.ds(r0, CH)], out_h.at[cid, pl.ds(r0, CH)])
            return carry

        lax.fori_loop(0, ROWS_PER_TILE // CH, obody, 0)

    return k(x, ea, src, dst, rel, zblk)


def _sc_edge_head(s_head, d_head, src, dst):
    k64 = 64

    @functools.partial(
        pl.kernel,
        out_type=jax.ShapeDtypeStruct((E, k64), jnp.float32),
        mesh=_mesh(),
        scratch_types=[
            pltpu.VMEM((CH,), jnp.int32),
            pltpu.VMEM((CH,), jnp.int32),
            pltpu.VMEM((CH, k64), jnp.float32),
            pltpu.VMEM((CH, k64), jnp.float32),
            pltpu.VMEM((REM,), jnp.int32),
            pltpu.VMEM((REM,), jnp.int32),
            pltpu.VMEM((REM, k64), jnp.float32),
            pltpu.VMEM((REM, k64), jnp.float32),
            pltpu.SemaphoreType.DMA,
            pltpu.SemaphoreType.DMA,
        ],
        compiler_params=pltpu.CompilerParams(use_tc_tiling_on_sc=False),
    )
    def k(s_h, d_h, src_h, dst_h, out_h,
          si, di, sv, dv, si2, di2, sv2, dv2, sem_a, sem_b):
        base = _wid() * EPW

        def do_chunk(off, n, si_, di_, sv_, dv_):
            pltpu.sync_copy(src_h.at[pl.ds(off, n)], si_)
            pltpu.sync_copy(dst_h.at[pl.ds(off, n)], di_)
            ca = pltpu.async_copy(s_h.at[si_], sv_, sem_a)
            cb = pltpu.async_copy(d_h.at[di_], dv_, sem_b)
            ca.wait()
            cb.wait()

            def cbody(i, carry):
                for jj in range(k64 // 16):
                    sl = pl.ds(jj * 16, 16)
                    sv_[i, sl] = sv_[i, sl] + dv_[i, sl]
                return carry

            lax.fori_loop(0, n, cbody, 0)
            pltpu.sync_copy(sv_, out_h.at[pl.ds(off, n)])

        def ebody(j, carry):
            do_chunk(pl.multiple_of(base + j * CH, 8), CH, si, di, sv, dv)
            return carry

        lax.fori_loop(0, NFULL, ebody, 0)
        do_chunk(pl.multiple_of(base + NFULL * CH, 8), REM, si2, di2, sv2, dv2)

    return k(s_head, d_head, src, dst)


def _tc_dense(x, adds, wt, b, do_norm, do_relu, blk):
    bb, d = x.shape
    kk = wt.shape[1]
    na = len(adds)

    def body(*refs):
        xv = refs[0][...]
        for a in refs[1:1 + na]:
            xv = xv + a[...]
        if do_norm:
            s = jnp.sum(xv * xv, axis=1, keepdims=True)
            xv = xv / jnp.maximum(jnp.sqrt(s), 1e-12)
        y = jnp.dot(xv, refs[1 + na][...], preferred_element_type=jnp.float32)
        y = y + refs[2 + na][...]
        if do_relu:
            y = jnp.maximum(y, 0.0)
        refs[3 + na][...] = y

    in_specs = [pl.BlockSpec((blk, d), lambda i: (i, 0))] * (1 + na) + [
        pl.BlockSpec((d, kk), lambda i: (0, 0)),
        pl.BlockSpec((1, kk), lambda i: (0, 0)),
    ]
    return pl.pallas_call(
        body,
        grid=(bb // blk,),
        in_specs=in_specs,
        out_specs=pl.BlockSpec((blk, kk), lambda i: (i, 0)),
        out_shape=jax.ShapeDtypeStruct((bb, kk), jnp.float32),
    )(x, *adds, wt, b)


def kernel(node_ids, rel_ids, center_mol_idx, non_molecule_node_ids, edge_index,
           node_emb_table, rel_emb_table, lin_W, lin_b,
           conv1_W, conv1_b, conv2_W, conv2_b,
           ec_W, ec_b, mp_W, mp_b, nc_W, nc_b, bp_W, bp_b):
    f32 = jnp.float32
    i32 = jnp.int32
    node_ids = node_ids.astype(i32)
    rel_ids = rel_ids.astype(i32)
    src = edge_index[0].astype(i32)
    dst = edge_index[1].astype(i32)

    ids_pad = jnp.concatenate([node_ids, jnp.zeros((NP - N,), i32)])
    rows = _sc_gather_rows(node_emb_table.astype(f32), ids_pad, 64)

    lin_bt = lin_b.reshape(1, -1)
    x0 = _tc_dense(rows, (), lin_W.T, lin_bt, True, True, 512)
    ea_u = _tc_dense(rel_emb_table.astype(f32), (), lin_W.T, lin_bt, True, True, 64)

    zblk = jnp.zeros((CH, D), f32)
    agg1 = _sc_msgpass(x0, ea_u, src, dst, rel_ids, zblk)
    x1 = _tc_dense(x0, (agg1[0], agg1[1]), conv1_W.T, conv1_b.reshape(1, -1),
                   False, True, 512)
    agg2 = _sc_msgpass(x1, ea_u, src, dst, rel_ids, zblk)
    x2 = _tc_dense(x1, (agg2[0], agg2[1]), conv2_W.T, conv2_b.reshape(1, -1),
                   False, False, 512)

    wcat = jnp.concatenate(
        [ec_W[:, :D].T, ec_W[:, D:].T, bp_W.T, jnp.zeros((D, 127), f32)], axis=1)
    bcat = jnp.concatenate(
        [ec_b, jnp.zeros((64,), f32), bp_b, jnp.zeros((127,), f32)]).reshape(1, -1)
    heads = _tc_dense(x2, (), wcat, bcat, False, False, 512)
    s_head = heads[:, :64]
    d_head = heads[:, 64:128]
    binary_pred = heads[:N, 128:129]

    edge_class = _sc_edge_head(s_head, d_head, src, dst)

    gidx = jnp.concatenate([center_mol_idx.astype(i32),
                            non_molecule_node_ids.astype(i32)])
    xg = _sc_gather_rows(x2, gidx, 96)
    mp_wt = jnp.pad(mp_W.T, ((0, 0), (0, 28)))
    mp_bp = jnp.pad(mp_b, (0, 28)).reshape(1, -1)
    motif_pred = _tc_dense(xg[:1024], (), mp_wt, mp_bp, False, False, 512)[:, :100]
    nc_wt = jnp.pad(nc_W.T, ((0, 0), (0, 113)))
    nc_bp = jnp.pad(nc_b, (0, 113)).reshape(1, -1)
    node_class = _tc_dense(xg[1024:], (), nc_wt, nc_bp, False, False, 512)[:, :15]

    return (edge_class, motif_pred, node_class, binary_pred)
